# Initial kernel scaffold; baseline (speedup 1.0000x reference)
import jax
import jax.numpy as jnp
from jax.experimental import pallas as pl


def kernel(x, basis, comp, root, bias, Wfc, edge_index, edge_type):
    raise NotImplementedError("write your pallas kernel here")


# speedup vs baseline: 8.4023x; 8.4023x over previous
import functools

import jax
import jax.numpy as jnp
from jax import lax
from jax.experimental import pallas as pl
from jax.experimental.pallas import tpu as pltpu
from jax.experimental.pallas import tpu_sc as plsc

N_NODES = 5000
N_USERS = 3000
N_REL = 5
N_BASES = 30
H0 = 64
H1 = 32
N_EDGES = 160000
N_SLAB = 1 + N_REL

E_CHUNK = 128
CNT_W = 8
PAD_E = 163840
A_ROWS = 25088

_NC = 2
_NS = 16
_NW = _NC * _NS
_ROWS_PER_SUB = A_ROWS // _NS
_CHUNKS_PER_W = PAD_E // E_CHUNK // _NW


def _wcat_body(comp_ref, basis_ref, root_ref, out_ref):
    parts = [root_ref[...]]
    for r in range(N_REL):
        acc = comp_ref[r, 0] * basis_ref[0]
        for b in range(1, N_BASES):
            acc = acc + comp_ref[r, b] * basis_ref[b]
        parts.append(acc)
    out_ref[...] = jnp.concatenate(parts, axis=1)


def _build_wcat(comp, basis, root):
    BA = 1000
    grid = N_NODES // BA
    return pl.pallas_call(
        _wcat_body,
        grid=(grid,),
        in_specs=[
            pl.BlockSpec(memory_space=pltpu.SMEM),
            pl.BlockSpec((N_BASES, BA, H0), lambda i: (0, i, 0)),
            pl.BlockSpec((BA, H0), lambda i: (i, 0)),
        ],
        out_specs=pl.BlockSpec((BA, N_SLAB * H0), lambda i: (i, 0)),
        out_shape=jax.ShapeDtypeStruct((N_NODES, N_SLAB * H0), jnp.float32),
    )(comp, basis, root)


def _mm_body(x_ref, w_ref, out_ref):
    out_ref[...] = jnp.dot(x_ref[...], w_ref[...],
                           preferred_element_type=jnp.float32)


def _build_h(x, wcat):
    BI = 256
    grid = pl.cdiv(N_NODES, BI)
    return pl.pallas_call(
        _mm_body,
        grid=(grid,),
        in_specs=[
            pl.BlockSpec((BI, N_NODES), lambda i: (i, 0)),
            pl.BlockSpec((N_NODES, N_SLAB * H0), lambda i: (0, 0)),
        ],
        out_specs=pl.BlockSpec((BI, N_SLAB * H0), lambda i: (i, 0)),
        out_shape=jax.ShapeDtypeStruct((N_NODES, N_SLAB * H0), jnp.float32),
    )(x, wcat)


def _sc_edge_body(htab, srcp, dstp, typp, zer2, zer1, ones_h,
                  a_out, cnt_out,
                  srcv, dstv, typv, gidxv, binv, onesv, rows,
                  a_sh, cnt_sh, sem):
    c = lax.axis_index("c")
    s = lax.axis_index("s")
    w = s * _NC + c

    pltpu.sync_copy(ones_h, onesv)

    off = s * _ROWS_PER_SUB
    pltpu.sync_copy(zer2.at[pl.ds(off, _ROWS_PER_SUB)],
                    a_sh.at[pl.ds(off, _ROWS_PER_SUB)])
    pltpu.sync_copy(zer1.at[pl.ds(off, _ROWS_PER_SUB)],
                    cnt_sh.at[pl.ds(off, _ROWS_PER_SUB)])
    plsc.subcore_barrier()

    def chunk_body(t, carry):
        base = (w + _NW * t) * E_CHUNK
        pltpu.sync_copy(srcp.at[pl.ds(base, E_CHUNK)], srcv)
        pltpu.sync_copy(dstp.at[pl.ds(base, E_CHUNK)], dstv)
        pltpu.sync_copy(typp.at[pl.ds(base, E_CHUNK)], typv)
        for j in range(E_CHUNK // 16):
            sl = pl.ds(j * 16, 16)
            tt = typv[sl]
            gidxv[sl] = srcv[sl] * N_SLAB + (tt + 1)
            binv[sl] = dstv[sl] * N_REL + tt
        pltpu.async_copy(htab.at[gidxv], rows, sem).wait()
        pltpu.sync_copy(rows, a_sh.at[binv], add=True)
        pltpu.sync_copy(onesv, cnt_sh.at[binv], add=True)
        return carry

    lax.fori_loop(0, _CHUNKS_PER_W, chunk_body, 0)
    plsc.subcore_barrier()

    pltpu.sync_copy(a_sh.at[pl.ds(off, _ROWS_PER_SUB)],
                    a_out.at[c, pl.ds(off, _ROWS_PER_SUB)])
    pltpu.sync_copy(cnt_sh.at[pl.ds(off, _ROWS_PER_SUB)],
                    cnt_out.at[c, pl.ds(off, _ROWS_PER_SUB)])


def _sc_aggregate(htab, srcp, dstp, typp):
    zer2 = jnp.zeros((A_ROWS, H0), jnp.float32)
    zer1 = jnp.zeros((A_ROWS, CNT_W), jnp.float32)
    ones_h = jnp.ones((E_CHUNK, CNT_W), jnp.float32)
    mesh = plsc.VectorSubcoreMesh(core_axis_name="c", subcore_axis_name="s")
    f = pl.kernel(
        _sc_edge_body,
        out_type=(
            jax.ShapeDtypeStruct((_NC, A_ROWS, H0), jnp.float32),
            jax.ShapeDtypeStruct((_NC, A_ROWS, CNT_W), jnp.float32),
        ),
        mesh=mesh,
        compiler_params=pltpu.CompilerParams(use_tc_tiling_on_sc=False),
        scratch_types=[
            pltpu.VMEM((E_CHUNK,), jnp.int32),
            pltpu.VMEM((E_CHUNK,), jnp.int32),
            pltpu.VMEM((E_CHUNK,), jnp.int32),
            pltpu.VMEM((E_CHUNK,), jnp.int32),
            pltpu.VMEM((E_CHUNK,), jnp.int32),
            pltpu.VMEM((E_CHUNK, CNT_W), jnp.float32),
            pltpu.VMEM((E_CHUNK, H0), jnp.float32),
            pltpu.VMEM_SHARED((A_ROWS, H0), jnp.float32),
            pltpu.VMEM_SHARED((A_ROWS, CNT_W), jnp.float32),
            pltpu.SemaphoreType.DMA,
        ],
    )
    return f(htab, srcp, dstp, typp, zer2, zer1, ones_h)


def _head_body(h_ref, a_ref, cnt_ref, bias_ref, wfc_ref, out_ref):
    bn = h_ref.shape[0]
    a = a_ref[0] + a_ref[1]
    cnt = cnt_ref[0] + cnt_ref[1]
    scaled = a / jnp.maximum(cnt, 1.0)
    g = jnp.sum(scaled.reshape(bn, N_REL, H0), axis=1)
    out = h_ref[:, 0:H0] + bias_ref[...] + g
    y = lax.dot_general(out, wfc_ref[...], (((1,), (1,)), ((), ())),
                        preferred_element_type=jnp.float32)
    out_ref[...] = jnp.maximum(y, 0.0)


def _head(h, a_bins, cnt_bins, bias, wfc):
    BN = 1000
    grid = N_NODES // BN
    return pl.pallas_call(
        _head_body,
        grid=(grid,),
        in_specs=[
            pl.BlockSpec((BN, N_SLAB * H0), lambda i: (i, 0)),
            pl.BlockSpec((_NC, BN * N_REL, H0), lambda i: (0, i, 0)),
            pl.BlockSpec((_NC, BN * N_REL, 1), lambda i: (0, i, 0)),
            pl.BlockSpec((1, H0), lambda i: (0, 0)),
            pl.BlockSpec((H1, H0), lambda i: (0, 0)),
        ],
        out_specs=pl.BlockSpec((BN, H1), lambda i: (i, 0)),
        out_shape=jax.ShapeDtypeStruct((N_NODES, H1), jnp.float32),
    )(h, a_bins, cnt_bins, bias, wfc)


def kernel(x, basis, comp, root, bias, Wfc, edge_index, edge_type):
    pad = PAD_E - N_EDGES
    ar = jnp.arange(pad, dtype=jnp.int32)
    src_p = jnp.concatenate([edge_index[0], (ar * 131) % N_NODES])
    dst_p = jnp.concatenate([edge_index[1], N_NODES + (ar % 17)])
    typ_p = jnp.concatenate([edge_type, jnp.zeros((pad,), jnp.int32)])

    wcat = _build_wcat(comp, basis, root)
    h = _build_h(x, wcat)
    htab = h.reshape(N_NODES * N_SLAB, H0)

    a_out, cnt_out = _sc_aggregate(htab, src_p, dst_p, typ_p)
    a_bins = a_out[:, : N_NODES * N_REL, :]
    cnt_bins = cnt_out[:, : N_NODES * N_REL, 0:1]

    y = _head(h, a_bins, cnt_bins, bias.reshape(1, H0), Wfc)
    return (y[:N_USERS], y[N_USERS:])


# speedup vs baseline: 11.6363x; 1.3849x over previous
import functools

import jax
import jax.numpy as jnp
from jax import lax
from jax.experimental import pallas as pl
from jax.experimental.pallas import tpu as pltpu
from jax.experimental.pallas import tpu_sc as plsc

N_NODES = 5000
N_USERS = 3000
N_REL = 5
N_BASES = 30
H0 = 64
H1 = 32
N_EDGES = 160000
N_SLAB = 1 + N_REL

E_CHUNK = 128
CNT_W = 8
PAD_E = 163840
A_ROWS = 25088

_NC = 2
_NS = 16
_NW = _NC * _NS
_ROWS_PER_SUB = A_ROWS // _NS
_CHUNKS_PER_W = PAD_E // E_CHUNK // _NW
_E_PER_W = PAD_E // _NW


def _wcat_body(comp_ref, basis_ref, root_ref, out_ref):
    parts = [root_ref[...]]
    for r in range(N_REL):
        acc = comp_ref[r, 0] * basis_ref[0]
        for b in range(1, N_BASES):
            acc = acc + comp_ref[r, b] * basis_ref[b]
        parts.append(acc)
    out_ref[...] = jnp.concatenate(parts, axis=1)


def _build_wcat(comp, basis, root):
    BA = 1000
    grid = N_NODES // BA
    return pl.pallas_call(
        _wcat_body,
        grid=(grid,),
        in_specs=[
            pl.BlockSpec(memory_space=pltpu.SMEM),
            pl.BlockSpec((N_BASES, BA, H0), lambda i: (0, i, 0)),
            pl.BlockSpec((BA, H0), lambda i: (i, 0)),
        ],
        out_specs=pl.BlockSpec((BA, N_SLAB * H0), lambda i: (i, 0)),
        out_shape=jax.ShapeDtypeStruct((N_NODES, N_SLAB * H0), jnp.float32),
    )(comp, basis, root)


def _mm_body(x_ref, w_ref, out_ref):
    out_ref[...] = jnp.dot(x_ref[...], w_ref[...],
                           preferred_element_type=jnp.float32)


def _build_h(x, wcat):
    BI = 256
    grid = pl.cdiv(N_NODES, BI)
    return pl.pallas_call(
        _mm_body,
        grid=(grid,),
        in_specs=[
            pl.BlockSpec((BI, N_NODES), lambda i: (i, 0)),
            pl.BlockSpec((N_NODES, N_SLAB * H0), lambda i: (0, 0)),
        ],
        out_specs=pl.BlockSpec((BI, N_SLAB * H0), lambda i: (i, 0)),
        out_shape=jax.ShapeDtypeStruct((N_NODES, N_SLAB * H0), jnp.float32),
    )(x, wcat)


_E_HALF = _E_PER_W // 2


def _sc_edge_body(htab, srcp, dstp, typp, zer2,
                  a_out,
                  srcv, dstv, typv, gidx0, gidx1, bin0, bin1,
                  rows0, rows1, sem0, sem1,
                  a_sh):
    c = lax.axis_index("c")
    s = lax.axis_index("s")
    w = s * _NC + c
    gidx = (gidx0, gidx1)
    binb = (bin0, bin1)
    rows = (rows0, rows1)
    sems = (sem0, sem1)

    off = s * _ROWS_PER_SUB
    pltpu.sync_copy(zer2.at[pl.ds(off, _ROWS_PER_SUB)],
                    a_sh.at[pl.ds(off, _ROWS_PER_SUB)])
    plsc.subcore_barrier()

    def compute_idx(hc, b):
        cb = hc * E_CHUNK
        for j in range(E_CHUNK // 16):
            sl = pl.ds(cb + j * 16, 16)
            ol = pl.ds(j * 16, 16)
            tt = typv[sl]
            gidx[b][ol] = srcv[sl] * N_SLAB + (tt + 1)
            binb[b][ol] = dstv[sl] * N_REL + tt

    half_chunks = _E_HALF // E_CHUNK
    for half in range(2):
        ebase = w * _E_PER_W + half * _E_HALF
        pltpu.sync_copy(srcp.at[pl.ds(ebase, _E_HALF)], srcv)
        pltpu.sync_copy(dstp.at[pl.ds(ebase, _E_HALF)], dstv)
        pltpu.sync_copy(typp.at[pl.ds(ebase, _E_HALF)], typv)

        for b in range(2):
            compute_idx(b, b)
            pltpu.async_copy(htab.at[gidx[b]], rows[b], sems[b])

        def chunk_body(t, carry):
            for b in range(2):
                hc = t * 2 + b
                pltpu.make_async_copy(htab.at[gidx[b]], rows[b], sems[b]).wait()
                pltpu.sync_copy(rows[b], a_sh.at[binb[b]], add=True)

                @pl.when(hc + 2 < half_chunks)
                def _():
                    compute_idx(hc + 2, b)
                    pltpu.async_copy(htab.at[gidx[b]], rows[b], sems[b])
            return carry

        lax.fori_loop(0, half_chunks // 2, chunk_body, 0)

    plsc.subcore_barrier()
    pltpu.sync_copy(a_sh.at[pl.ds(off, _ROWS_PER_SUB)],
                    a_out.at[c, pl.ds(off, _ROWS_PER_SUB)])


def _sc_cnt_body(dstp, typp, zer1, ones_h,
                 cnt_out,
                 dstv, typv, binv, onesv,
                 cnt_sh):
    c = lax.axis_index("c")
    s = lax.axis_index("s")
    w = s * _NC + c

    pltpu.sync_copy(ones_h, onesv)
    ebase = w * _E_PER_W
    pltpu.sync_copy(dstp.at[pl.ds(ebase, _E_PER_W)], dstv)
    pltpu.sync_copy(typp.at[pl.ds(ebase, _E_PER_W)], typv)

    off = s * _ROWS_PER_SUB
    pltpu.sync_copy(zer1.at[pl.ds(off, _ROWS_PER_SUB)],
                    cnt_sh.at[pl.ds(off, _ROWS_PER_SUB)])
    plsc.subcore_barrier()

    def chunk_body(t, carry):
        cb = t * E_CHUNK
        for j in range(E_CHUNK // 16):
            sl = pl.ds(cb + j * 16, 16)
            ol = pl.ds(j * 16, 16)
            binv[ol] = dstv[sl] * N_REL + typv[sl]
        pltpu.sync_copy(onesv, cnt_sh.at[binv], add=True)
        return carry

    lax.fori_loop(0, _CHUNKS_PER_W, chunk_body, 0)
    plsc.subcore_barrier()
    pltpu.sync_copy(cnt_sh.at[pl.ds(off, _ROWS_PER_SUB)],
                    cnt_out.at[c, pl.ds(off, _ROWS_PER_SUB)])


def _sc_aggregate(htab, srcp, dstp, typp):
    zer2 = jnp.zeros((A_ROWS, H0), jnp.float32)
    mesh = plsc.VectorSubcoreMesh(core_axis_name="c", subcore_axis_name="s")
    f = pl.kernel(
        _sc_edge_body,
        out_type=jax.ShapeDtypeStruct((_NC, A_ROWS, H0), jnp.float32),
        mesh=mesh,
        compiler_params=pltpu.CompilerParams(use_tc_tiling_on_sc=False),
        scratch_types=[
            pltpu.VMEM((_E_HALF,), jnp.int32),
            pltpu.VMEM((_E_HALF,), jnp.int32),
            pltpu.VMEM((_E_HALF,), jnp.int32),
            pltpu.VMEM((E_CHUNK,), jnp.int32),
            pltpu.VMEM((E_CHUNK,), jnp.int32),
            pltpu.VMEM((E_CHUNK,), jnp.int32),
            pltpu.VMEM((E_CHUNK,), jnp.int32),
            pltpu.VMEM((E_CHUNK, H0), jnp.float32),
            pltpu.VMEM((E_CHUNK, H0), jnp.float32),
            pltpu.SemaphoreType.DMA,
            pltpu.SemaphoreType.DMA,
            pltpu.VMEM_SHARED((A_ROWS, H0), jnp.float32),
        ],
    )
    return f(htab, srcp, dstp, typp, zer2)


def _sc_counts(dstp, typp):
    zer1 = jnp.zeros((A_ROWS, CNT_W), jnp.float32)
    ones_h = jnp.ones((E_CHUNK, CNT_W), jnp.float32)
    mesh = plsc.VectorSubcoreMesh(core_axis_name="c", subcore_axis_name="s")
    f = pl.kernel(
        _sc_cnt_body,
        out_type=jax.ShapeDtypeStruct((_NC, A_ROWS, CNT_W), jnp.float32),
        mesh=mesh,
        compiler_params=pltpu.CompilerParams(use_tc_tiling_on_sc=False),
        scratch_types=[
            pltpu.VMEM((_E_PER_W,), jnp.int32),
            pltpu.VMEM((_E_PER_W,), jnp.int32),
            pltpu.VMEM((E_CHUNK,), jnp.int32),
            pltpu.VMEM((E_CHUNK, CNT_W), jnp.float32),
            pltpu.VMEM_SHARED((A_ROWS, CNT_W), jnp.float32),
        ],
    )
    return f(dstp, typp, zer1, ones_h)


def _head_body(h_ref, a_ref, cnt_ref, bias_ref, wfc_ref, out_ref):
    bn = h_ref.shape[0]
    a = a_ref[0] + a_ref[1]
    cnt = cnt_ref[0, :, 0:1] + cnt_ref[1, :, 0:1]
    scaled = a / jnp.maximum(cnt, 1.0)
    g = jnp.sum(scaled.reshape(bn, N_REL, H0), axis=1)
    out = h_ref[:, 0:H0] + bias_ref[...] + g
    y = lax.dot_general(out, wfc_ref[...], (((1,), (1,)), ((), ())),
                        preferred_element_type=jnp.float32)
    out_ref[...] = jnp.maximum(y, 0.0)


def _head(h, a_bins, cnt_bins, bias, wfc):
    BN = 1000
    grid = N_NODES // BN
    return pl.pallas_call(
        _head_body,
        grid=(grid,),
        in_specs=[
            pl.BlockSpec((BN, N_SLAB * H0), lambda i: (i, 0)),
            pl.BlockSpec((_NC, BN * N_REL, H0), lambda i: (0, i, 0)),
            pl.BlockSpec((_NC, BN * N_REL, CNT_W), lambda i: (0, i, 0)),
            pl.BlockSpec((1, H0), lambda i: (0, 0)),
            pl.BlockSpec((H1, H0), lambda i: (0, 0)),
        ],
        out_specs=pl.BlockSpec((BN, H1), lambda i: (i, 0)),
        out_shape=jax.ShapeDtypeStruct((N_NODES, H1), jnp.float32),
    )(h, a_bins, cnt_bins, bias, wfc)


def kernel(x, basis, comp, root, bias, Wfc, edge_index, edge_type):
    pad = PAD_E - N_EDGES
    ar = jnp.arange(pad, dtype=jnp.int32)
    src_p = jnp.concatenate([edge_index[0], (ar * 131) % N_NODES])
    dst_p = jnp.concatenate([edge_index[1], N_NODES + (ar % 17)])
    typ_p = jnp.concatenate([edge_type, jnp.zeros((pad,), jnp.int32)])

    wcat = _build_wcat(comp, basis, root)
    h = _build_h(x, wcat)
    htab = h.reshape(N_NODES * N_SLAB, H0)

    cnt_out = _sc_counts(dst_p, typ_p)
    a_out = _sc_aggregate(htab, src_p, dst_p, typ_p)
    y = _head(h, a_out, cnt_out, bias.reshape(1, H0), Wfc)
    return (y[:N_USERS], y[N_USERS:])


# speedup vs baseline: 11.7861x; 1.0129x over previous
import functools

import jax
import jax.numpy as jnp
from jax import lax
from jax.experimental import pallas as pl
from jax.experimental.pallas import tpu as pltpu
from jax.experimental.pallas import tpu_sc as plsc

N_NODES = 5000
N_USERS = 3000
N_REL = 5
N_BASES = 30
H0 = 64
H1 = 32
N_EDGES = 160000
N_SLAB = 1 + N_REL

E_CHUNK = 128
CNT_W = 8
PAD_E = 163840
NODE_PAD = 5120
A_ROWS = N_REL * NODE_PAD

_NC = 2
_NS = 16
_NW = _NC * _NS
_ROWS_PER_SUB = A_ROWS // _NS
_CHUNKS_PER_W = PAD_E // E_CHUNK // _NW
_E_PER_W = PAD_E // _NW


def _wcat_body(comp_ref, basis_ref, root_ref, out_ref):
    accs = [None] * N_REL
    for b in range(N_BASES):
        v = basis_ref[b]
        for r in range(N_REL):
            t = comp_ref[r, b] * v
            accs[r] = t if accs[r] is None else accs[r] + t
    out_ref[...] = jnp.concatenate([root_ref[...]] + accs, axis=1)


def _build_wcat(comp, basis, root):
    BA = 1000
    grid = N_NODES // BA
    return pl.pallas_call(
        _wcat_body,
        grid=(grid,),
        in_specs=[
            pl.BlockSpec(memory_space=pltpu.SMEM),
            pl.BlockSpec((N_BASES, BA, H0), lambda i: (0, i, 0)),
            pl.BlockSpec((BA, H0), lambda i: (i, 0)),
        ],
        out_specs=pl.BlockSpec((BA, N_SLAB * H0), lambda i: (i, 0)),
        out_shape=jax.ShapeDtypeStruct((N_NODES, N_SLAB * H0), jnp.float32),
    )(comp, basis, root)


def _mm_body(x_ref, w_ref, out_ref):
    out_ref[...] = jnp.dot(x_ref[...], w_ref[...],
                           preferred_element_type=jnp.float32)


def _build_h(x, wcat):
    BI = 256
    grid = pl.cdiv(N_NODES, BI)
    return pl.pallas_call(
        _mm_body,
        grid=(grid,),
        in_specs=[
            pl.BlockSpec((BI, N_NODES), lambda i: (i, 0)),
            pl.BlockSpec((N_NODES, N_SLAB * H0), lambda i: (0, 0)),
        ],
        out_specs=pl.BlockSpec((BI, N_SLAB * H0), lambda i: (i, 0)),
        out_shape=jax.ShapeDtypeStruct((N_NODES, N_SLAB * H0), jnp.float32),
    )(x, wcat)


_E_HALF = _E_PER_W // 2


def _sc_edge_body(htab, srcp, dstp, typp, zer2,
                  a_out,
                  srcv, dstv, typv, gidx0, gidx1, bin0, bin1,
                  rows0, rows1, sem0, sem1,
                  a_sh):
    c = lax.axis_index("c")
    s = lax.axis_index("s")
    w = s * _NC + c
    gidx = (gidx0, gidx1)
    binb = (bin0, bin1)
    rows = (rows0, rows1)
    sems = (sem0, sem1)

    off = s * _ROWS_PER_SUB
    pltpu.sync_copy(zer2.at[pl.ds(off, _ROWS_PER_SUB)],
                    a_sh.at[pl.ds(off, _ROWS_PER_SUB)])
    plsc.subcore_barrier()

    def compute_idx(hc, b):
        cb = hc * E_CHUNK
        for j in range(E_CHUNK // 16):
            sl = pl.ds(cb + j * 16, 16)
            ol = pl.ds(j * 16, 16)
            tt = typv[sl]
            gidx[b][ol] = srcv[sl] * N_SLAB + (tt + 1)
            binb[b][ol] = tt * NODE_PAD + dstv[sl]

    half_chunks = _E_HALF // E_CHUNK
    for half in range(2):
        ebase = w * _E_PER_W + half * _E_HALF
        pltpu.sync_copy(srcp.at[pl.ds(ebase, _E_HALF)], srcv)
        pltpu.sync_copy(dstp.at[pl.ds(ebase, _E_HALF)], dstv)
        pltpu.sync_copy(typp.at[pl.ds(ebase, _E_HALF)], typv)

        for b in range(2):
            compute_idx(b, b)
            pltpu.async_copy(htab.at[gidx[b]], rows[b], sems[b])

        def chunk_body(t, carry):
            for b in range(2):
                hc = t * 2 + b
                pltpu.make_async_copy(htab.at[gidx[b]], rows[b], sems[b]).wait()
                pltpu.sync_copy(rows[b], a_sh.at[binb[b]], add=True)

                @pl.when(hc + 2 < half_chunks)
                def _():
                    compute_idx(hc + 2, b)
                    pltpu.async_copy(htab.at[gidx[b]], rows[b], sems[b])
            return carry

        lax.fori_loop(0, half_chunks // 2, chunk_body, 0)

    plsc.subcore_barrier()
    pltpu.sync_copy(a_sh.at[pl.ds(off, _ROWS_PER_SUB)],
                    a_out.at[c, pl.ds(off, _ROWS_PER_SUB)])


def _sc_cnt_body(dstp, typp, zer1, ones_h,
                 cnt_out,
                 dstv, typv, binv, onesv,
                 cnt_sh):
    c = lax.axis_index("c")
    s = lax.axis_index("s")
    w = s * _NC + c

    pltpu.sync_copy(ones_h, onesv)
    ebase = w * _E_PER_W
    pltpu.sync_copy(dstp.at[pl.ds(ebase, _E_PER_W)], dstv)
    pltpu.sync_copy(typp.at[pl.ds(ebase, _E_PER_W)], typv)

    off = s * _ROWS_PER_SUB
    pltpu.sync_copy(zer1.at[pl.ds(off, _ROWS_PER_SUB)],
                    cnt_sh.at[pl.ds(off, _ROWS_PER_SUB)])
    plsc.subcore_barrier()

    def chunk_body(t, carry):
        cb = t * E_CHUNK
        for j in range(E_CHUNK // 16):
            sl = pl.ds(cb + j * 16, 16)
            ol = pl.ds(j * 16, 16)
            binv[ol] = typv[sl] * NODE_PAD + dstv[sl]
        pltpu.sync_copy(onesv, cnt_sh.at[binv], add=True)
        return carry

    lax.fori_loop(0, _CHUNKS_PER_W, chunk_body, 0)
    plsc.subcore_barrier()
    pltpu.sync_copy(cnt_sh.at[pl.ds(off, _ROWS_PER_SUB)],
                    cnt_out.at[c, pl.ds(off, _ROWS_PER_SUB)])


def _sc_aggregate(htab, srcp, dstp, typp):
    zer2 = jnp.zeros((A_ROWS, H0), jnp.float32)
    mesh = plsc.VectorSubcoreMesh(core_axis_name="c", subcore_axis_name="s")
    f = pl.kernel(
        _sc_edge_body,
        out_type=jax.ShapeDtypeStruct((_NC, A_ROWS, H0), jnp.float32),
        mesh=mesh,
        compiler_params=pltpu.CompilerParams(use_tc_tiling_on_sc=False),
        scratch_types=[
            pltpu.VMEM((_E_HALF,), jnp.int32),
            pltpu.VMEM((_E_HALF,), jnp.int32),
            pltpu.VMEM((_E_HALF,), jnp.int32),
            pltpu.VMEM((E_CHUNK,), jnp.int32),
            pltpu.VMEM((E_CHUNK,), jnp.int32),
            pltpu.VMEM((E_CHUNK,), jnp.int32),
            pltpu.VMEM((E_CHUNK,), jnp.int32),
            pltpu.VMEM((E_CHUNK, H0), jnp.float32),
            pltpu.VMEM((E_CHUNK, H0), jnp.float32),
            pltpu.SemaphoreType.DMA,
            pltpu.SemaphoreType.DMA,
            pltpu.VMEM_SHARED((A_ROWS, H0), jnp.float32),
        ],
    )
    return f(htab, srcp, dstp, typp, zer2)


def _sc_counts(dstp, typp):
    zer1 = jnp.zeros((A_ROWS, CNT_W), jnp.float32)
    ones_h = jnp.ones((E_CHUNK, CNT_W), jnp.float32)
    mesh = plsc.VectorSubcoreMesh(core_axis_name="c", subcore_axis_name="s")
    f = pl.kernel(
        _sc_cnt_body,
        out_type=jax.ShapeDtypeStruct((_NC, A_ROWS, CNT_W), jnp.float32),
        mesh=mesh,
        compiler_params=pltpu.CompilerParams(use_tc_tiling_on_sc=False),
        scratch_types=[
            pltpu.VMEM((_E_PER_W,), jnp.int32),
            pltpu.VMEM((_E_PER_W,), jnp.int32),
            pltpu.VMEM((E_CHUNK,), jnp.int32),
            pltpu.VMEM((E_CHUNK, CNT_W), jnp.float32),
            pltpu.VMEM_SHARED((A_ROWS, CNT_W), jnp.float32),
        ],
    )
    return f(dstp, typp, zer1, ones_h)


def _head_body(h_ref, a_ref, cnt_ref, bias_ref, wfc_ref, out_ref):
    g = None
    for r in range(N_REL):
        ar = a_ref[0, r] + a_ref[1, r]
        cr = cnt_ref[0, r, :, 0:1] + cnt_ref[1, r, :, 0:1]
        sr = ar / jnp.maximum(cr, 1.0)
        g = sr if g is None else g + sr
    out = h_ref[:, 0:H0] + bias_ref[...] + g
    y = lax.dot_general(out, wfc_ref[...], (((1,), (1,)), ((), ())),
                        preferred_element_type=jnp.float32)
    out_ref[...] = jnp.maximum(y, 0.0)


def _head(h, a_bins, cnt_bins, bias, wfc):
    BN = 1000
    grid = N_NODES // BN
    return pl.pallas_call(
        _head_body,
        grid=(grid,),
        in_specs=[
            pl.BlockSpec((BN, N_SLAB * H0), lambda i: (i, 0)),
            pl.BlockSpec((_NC, N_REL, BN, H0), lambda i: (0, 0, i, 0)),
            pl.BlockSpec((_NC, N_REL, BN, CNT_W), lambda i: (0, 0, i, 0)),
            pl.BlockSpec((1, H0), lambda i: (0, 0)),
            pl.BlockSpec((H1, H0), lambda i: (0, 0)),
        ],
        out_specs=pl.BlockSpec((BN, H1), lambda i: (i, 0)),
        out_shape=jax.ShapeDtypeStruct((N_NODES, H1), jnp.float32),
    )(h, a_bins, cnt_bins, bias, wfc)


def kernel(x, basis, comp, root, bias, Wfc, edge_index, edge_type):
    pad = PAD_E - N_EDGES
    ar = jnp.arange(pad, dtype=jnp.int32)
    src_p = jnp.concatenate([edge_index[0], (ar * 131) % N_NODES])
    dst_p = jnp.concatenate([edge_index[1], N_NODES + (ar % (NODE_PAD - N_NODES))])
    typ_p = jnp.concatenate([edge_type, jnp.zeros((pad,), jnp.int32)])

    wcat = _build_wcat(comp, basis, root)
    h = _build_h(x, wcat)
    htab = h.reshape(N_NODES * N_SLAB, H0)

    cnt_out = _sc_counts(dst_p, typ_p)
    a_out = _sc_aggregate(htab, src_p, dst_p, typ_p)
    a4 = a_out.reshape(_NC, N_REL, NODE_PAD, H0)
    c4 = cnt_out.reshape(_NC, N_REL, NODE_PAD, CNT_W)
    y = _head(h, a4, c4, bias.reshape(1, H0), Wfc)
    return (y[:N_USERS], y[N_USERS:])


# speedup vs baseline: 14.5470x; 1.2342x over previous
import functools

import jax
import jax.numpy as jnp
from jax import lax
from jax.experimental import pallas as pl
from jax.experimental.pallas import tpu as pltpu
from jax.experimental.pallas import tpu_sc as plsc

N_NODES = 5000
N_USERS = 3000
N_REL = 5
N_BASES = 30
H0 = 64
H1 = 32
N_EDGES = 160000
N_SLAB = 1 + N_REL

E_CHUNK = 128
CNT_W = 8
PAD_E = 163840
NODE_PAD = 5120
A_ROWS = N_REL * NODE_PAD

_NC = 2
_NS = 16
_NW = _NC * _NS
_ROWS_PER_SUB = A_ROWS // _NS
_CHUNKS_PER_W = PAD_E // E_CHUNK // _NW
_E_PER_W = PAD_E // _NW


def _wcat_body(comp_ref, basist_ref, roott_ref, out_ref):
    ba = roott_ref.shape[1]
    out_ref[0:H0, :] = roott_ref[...]
    for ci in range(ba // 128):
        csl = pl.ds(ci * 128, 128)
        accs = [None] * N_REL
        for b in range(N_BASES):
            v = basist_ref[b, :, csl]
            for r in range(N_REL):
                t = comp_ref[r, b] * v
                accs[r] = t if accs[r] is None else accs[r] + t
        for r in range(N_REL):
            out_ref[H0 * (r + 1):H0 * (r + 2), csl] = accs[r]


def _build_wcat(comp, basist, roott):
    BA = 640
    grid = pl.cdiv(N_NODES, BA)
    return pl.pallas_call(
        _wcat_body,
        grid=(grid,),
        in_specs=[
            pl.BlockSpec(memory_space=pltpu.SMEM),
            pl.BlockSpec((N_BASES, H0, BA), lambda i: (0, 0, i)),
            pl.BlockSpec((H0, BA), lambda i: (0, i)),
        ],
        out_specs=pl.BlockSpec((N_SLAB * H0, BA), lambda i: (0, i)),
        out_shape=jax.ShapeDtypeStruct((N_SLAB * H0, N_NODES), jnp.float32),
    )(comp, basist, roott)


def _mm_body(x_ref, wt_ref, out_ref):
    out_ref[...] = lax.dot_general(x_ref[...], wt_ref[...],
                                   (((1,), (1,)), ((), ())),
                                   preferred_element_type=jnp.float32)


def _build_h(x, wcat_t):
    BI = 256
    grid = pl.cdiv(N_NODES, BI)
    return pl.pallas_call(
        _mm_body,
        grid=(grid,),
        in_specs=[
            pl.BlockSpec((BI, N_NODES), lambda i: (i, 0)),
            pl.BlockSpec((N_SLAB * H0, N_NODES), lambda i: (0, 0)),
        ],
        out_specs=pl.BlockSpec((BI, N_SLAB * H0), lambda i: (i, 0)),
        out_shape=jax.ShapeDtypeStruct((N_NODES, N_SLAB * H0), jnp.float32),
    )(x, wcat_t)


_E_HALF = _E_PER_W // 2


def _sc_edge_body(htab, srcp, dstp, typp, zer2,
                  a_out,
                  srcv, dstv, typv, gidx0, gidx1, bin0, bin1,
                  rows0, rows1, sem0, sem1,
                  a_sh):
    c = lax.axis_index("c")
    s = lax.axis_index("s")
    w = s * _NC + c
    gidx = (gidx0, gidx1)
    binb = (bin0, bin1)
    rows = (rows0, rows1)
    sems = (sem0, sem1)

    off = s * _ROWS_PER_SUB
    pltpu.sync_copy(zer2.at[pl.ds(off, _ROWS_PER_SUB)],
                    a_sh.at[pl.ds(off, _ROWS_PER_SUB)])
    plsc.subcore_barrier()

    def compute_idx(hc, b):
        cb = hc * E_CHUNK
        for j in range(E_CHUNK // 16):
            sl = pl.ds(cb + j * 16, 16)
            ol = pl.ds(j * 16, 16)
            tt = typv[sl]
            gidx[b][ol] = srcv[sl] * N_SLAB + (tt + 1)
            binb[b][ol] = tt * NODE_PAD + dstv[sl]

    half_chunks = _E_HALF // E_CHUNK
    for half in range(2):
        ebase = w * _E_PER_W + half * _E_HALF
        pltpu.sync_copy(srcp.at[pl.ds(ebase, _E_HALF)], srcv)
        pltpu.sync_copy(dstp.at[pl.ds(ebase, _E_HALF)], dstv)
        pltpu.sync_copy(typp.at[pl.ds(ebase, _E_HALF)], typv)

        for b in range(2):
            compute_idx(b, b)
            pltpu.async_copy(htab.at[gidx[b]], rows[b], sems[b])

        def chunk_body(t, carry):
            for b in range(2):
                hc = t * 2 + b
                pltpu.make_async_copy(htab.at[gidx[b]], rows[b], sems[b]).wait()
                pltpu.sync_copy(rows[b], a_sh.at[binb[b]], add=True)

                @pl.when(hc + 2 < half_chunks)
                def _():
                    compute_idx(hc + 2, b)
                    pltpu.async_copy(htab.at[gidx[b]], rows[b], sems[b])
            return carry

        lax.fori_loop(0, half_chunks // 2, chunk_body, 0)

    plsc.subcore_barrier()
    pltpu.sync_copy(a_sh.at[pl.ds(off, _ROWS_PER_SUB)],
                    a_out.at[c, pl.ds(off, _ROWS_PER_SUB)])


def _sc_cnt_body(dstp, typp, zer1, ones_h,
                 cnt_out,
                 dstv, typv, binv, onesv,
                 cnt_sh):
    c = lax.axis_index("c")
    s = lax.axis_index("s")
    w = s * _NC + c

    pltpu.sync_copy(ones_h, onesv)
    ebase = w * _E_PER_W
    pltpu.sync_copy(dstp.at[pl.ds(ebase, _E_PER_W)], dstv)
    pltpu.sync_copy(typp.at[pl.ds(ebase, _E_PER_W)], typv)

    off = s * _ROWS_PER_SUB
    pltpu.sync_copy(zer1.at[pl.ds(off, _ROWS_PER_SUB)],
                    cnt_sh.at[pl.ds(off, _ROWS_PER_SUB)])
    plsc.subcore_barrier()

    def chunk_body(t, carry):
        cb = t * E_CHUNK
        for j in range(E_CHUNK // 16):
            sl = pl.ds(cb + j * 16, 16)
            ol = pl.ds(j * 16, 16)
            binv[ol] = typv[sl] * NODE_PAD + dstv[sl]
        pltpu.sync_copy(onesv, cnt_sh.at[binv], add=True)
        return carry

    lax.fori_loop(0, _CHUNKS_PER_W, chunk_body, 0)
    plsc.subcore_barrier()
    pltpu.sync_copy(cnt_sh.at[pl.ds(off, _ROWS_PER_SUB)],
                    cnt_out.at[c, pl.ds(off, _ROWS_PER_SUB)])


_NORM_NB = NODE_PAD // _NW


def _sc_norm_body(a_hbm, cntf_hbm, g_out,
                  a0, a1, c0, c1, g):
    c = lax.axis_index("c")
    s = lax.axis_index("s")
    w = s * _NC + c
    nb = w * _NORM_NB

    for r in range(N_REL):
        rb = r * NODE_PAD + nb
        pltpu.sync_copy(a_hbm.at[0, pl.ds(rb, _NORM_NB)], a0)
        pltpu.sync_copy(a_hbm.at[1, pl.ds(rb, _NORM_NB)], a1)
        pltpu.sync_copy(cntf_hbm.at[0, pl.ds(rb * CNT_W, _NORM_NB * CNT_W)], c0)
        pltpu.sync_copy(cntf_hbm.at[1, pl.ds(rb * CNT_W, _NORM_NB * CNT_W)], c1)

        def inv_body(i, carry):
            sl = pl.ds(i * 16, 16)
            c0[sl] = 1.0 / jnp.maximum(c0[sl] + c1[sl], 1.0)
            return carry
        lax.fori_loop(0, _NORM_NB * CNT_W // 16, inv_body, 0)

        def row_body(n, carry):
            invv = plsc.load_gather(
                c0, [jnp.full((16,), n * CNT_W, jnp.int32)])
            for cg in range(H0 // 16):
                sl = pl.ds(cg * 16, 16)
                v = (a0[n, sl] + a1[n, sl]) * invv
                if r == 0:
                    g[n, sl] = v
                else:
                    g[n, sl] = g[n, sl] + v
            return carry
        lax.fori_loop(0, _NORM_NB, row_body, 0)

    pltpu.sync_copy(g, g_out.at[pl.ds(nb, _NORM_NB)])


def _sc_normalize(a_out, cnt_out):
    cntf = cnt_out.reshape(_NC, A_ROWS * CNT_W)
    mesh = plsc.VectorSubcoreMesh(core_axis_name="c", subcore_axis_name="s")
    f = pl.kernel(
        _sc_norm_body,
        out_type=jax.ShapeDtypeStruct((NODE_PAD, H0), jnp.float32),
        mesh=mesh,
        compiler_params=pltpu.CompilerParams(use_tc_tiling_on_sc=False,
                                             needs_layout_passes=False),
        scratch_types=[
            pltpu.VMEM((_NORM_NB, H0), jnp.float32),
            pltpu.VMEM((_NORM_NB, H0), jnp.float32),
            pltpu.VMEM((_NORM_NB * CNT_W,), jnp.float32),
            pltpu.VMEM((_NORM_NB * CNT_W,), jnp.float32),
            pltpu.VMEM((_NORM_NB, H0), jnp.float32),
        ],
    )
    return f(a_out, cntf)


def _sc_aggregate(htab, srcp, dstp, typp):
    zer2 = jnp.zeros((A_ROWS, H0), jnp.float32)
    mesh = plsc.VectorSubcoreMesh(core_axis_name="c", subcore_axis_name="s")
    f = pl.kernel(
        _sc_edge_body,
        out_type=jax.ShapeDtypeStruct((_NC, A_ROWS, H0), jnp.float32),
        mesh=mesh,
        compiler_params=pltpu.CompilerParams(use_tc_tiling_on_sc=False),
        scratch_types=[
            pltpu.VMEM((_E_HALF,), jnp.int32),
            pltpu.VMEM((_E_HALF,), jnp.int32),
            pltpu.VMEM((_E_HALF,), jnp.int32),
            pltpu.VMEM((E_CHUNK,), jnp.int32),
            pltpu.VMEM((E_CHUNK,), jnp.int32),
            pltpu.VMEM((E_CHUNK,), jnp.int32),
            pltpu.VMEM((E_CHUNK,), jnp.int32),
            pltpu.VMEM((E_CHUNK, H0), jnp.float32),
            pltpu.VMEM((E_CHUNK, H0), jnp.float32),
            pltpu.SemaphoreType.DMA,
            pltpu.SemaphoreType.DMA,
            pltpu.VMEM_SHARED((A_ROWS, H0), jnp.float32),
        ],
    )
    return f(htab, srcp, dstp, typp, zer2)


def _sc_counts(dstp, typp):
    zer1 = jnp.zeros((A_ROWS, CNT_W), jnp.float32)
    ones_h = jnp.ones((E_CHUNK, CNT_W), jnp.float32)
    mesh = plsc.VectorSubcoreMesh(core_axis_name="c", subcore_axis_name="s")
    f = pl.kernel(
        _sc_cnt_body,
        out_type=jax.ShapeDtypeStruct((_NC, A_ROWS, CNT_W), jnp.float32),
        mesh=mesh,
        compiler_params=pltpu.CompilerParams(use_tc_tiling_on_sc=False),
        scratch_types=[
            pltpu.VMEM((_E_PER_W,), jnp.int32),
            pltpu.VMEM((_E_PER_W,), jnp.int32),
            pltpu.VMEM((E_CHUNK,), jnp.int32),
            pltpu.VMEM((E_CHUNK, CNT_W), jnp.float32),
            pltpu.VMEM_SHARED((A_ROWS, CNT_W), jnp.float32),
        ],
    )
    return f(dstp, typp, zer1, ones_h)


def _head_body(h_ref, g_ref, bias_ref, wfc_ref, out_ref):
    out = h_ref[:, 0:H0] + bias_ref[...] + g_ref[...]
    y = lax.dot_general(out, wfc_ref[...], (((1,), (1,)), ((), ())),
                        preferred_element_type=jnp.float32)
    out_ref[...] = jnp.maximum(y, 0.0)


def _head(h, g, bias, wfc):
    BN = 1000
    grid = N_NODES // BN
    return pl.pallas_call(
        _head_body,
        grid=(grid,),
        in_specs=[
            pl.BlockSpec((BN, N_SLAB * H0), lambda i: (i, 0)),
            pl.BlockSpec((BN, H0), lambda i: (i, 0)),
            pl.BlockSpec((1, H0), lambda i: (0, 0)),
            pl.BlockSpec((H1, H0), lambda i: (0, 0)),
        ],
        out_specs=pl.BlockSpec((BN, H1), lambda i: (i, 0)),
        out_shape=jax.ShapeDtypeStruct((N_NODES, H1), jnp.float32),
    )(h, g, bias, wfc)


def kernel(x, basis, comp, root, bias, Wfc, edge_index, edge_type):
    pad = PAD_E - N_EDGES
    ar = jnp.arange(pad, dtype=jnp.int32)
    src_p = jnp.concatenate([edge_index[0], (ar * 131) % N_NODES])
    dst_p = jnp.concatenate([edge_index[1], N_NODES + (ar % (NODE_PAD - N_NODES))])
    typ_p = jnp.concatenate([edge_type, jnp.zeros((pad,), jnp.int32)])

    basist = jnp.transpose(basis, (0, 2, 1))
    roott = jnp.transpose(root)
    wcat_t = _build_wcat(comp, basist, roott)
    h = _build_h(x, wcat_t)
    htab = h.reshape(N_NODES * N_SLAB, H0)

    cnt_out = _sc_counts(dst_p, typ_p)
    a_out = _sc_aggregate(htab, src_p, dst_p, typ_p)
    g = _sc_normalize(a_out, cnt_out)
    y = _head(h, g, bias.reshape(1, H0), Wfc)
    return (y[:N_USERS], y[N_USERS:])
